# baseline (speedup 1.0000x reference)
import functools

import jax
import jax.numpy as jnp
from jax import lax
from jax.experimental import pallas as pl
from jax.experimental.pallas import tpu as pltpu
from jax.experimental.pallas import tpu_sc as plsc

_B = 4096
_L = 200
_D = 64
_NC = 2
_NS = 16
_NW = _NC * _NS
_RPW = _B // _NW
_SK = _RPW + 1


def _emb_body(idx_hbm, seq_hbm, table_hbm,
              out_hbm,
              idxl_v, idxt_v, seq_v, rows_a, rows_b, out_a, out_b,
              sem_g, sem_w0, sem_w1):
    wid = lax.axis_index("s") * _NC + lax.axis_index("c")
    b0 = wid * _RPW
    pltpu.sync_copy(seq_hbm.at[pl.ds(b0, _RPW)], seq_v.at[pl.ds(0, _RPW)])
    pltpu.sync_copy(idx_hbm.at[pl.ds(b0 * _L, _RPW * _L)], idxl_v)

    iota16 = lax.iota(jnp.int32, 16)

    def tbody(l, carry):
        lsplat = jnp.full((16,), 0, jnp.int32) + l
        for jg in range(8):
            flat = (iota16 + (jg * 16)) * _L + lsplat
            vals = plsc.load_gather(idxl_v, [flat])
            idxt_v[l, pl.ds(jg * 16, 16)] = vals
        return carry
    lax.fori_loop(0, _L, tbody, 0)

    pltpu.async_copy(table_hbm.at[idxt_v.at[0]], rows_a, sem_g)

    dbase = [iota16 + (c4 * 16) for c4 in range(4)]

    def do_l(l, rows_v, out_v, sem_w, nxt_rows, j):
        @pl.when(j >= 1)
        def _():
            pltpu.make_async_copy(out_hbm.at[0, :, pl.ds(0, _RPW)],
                                  out_v.at[:, pl.ds(0, _RPW)], sem_w).wait()
        pltpu.make_async_copy(table_hbm.at[pl.ds(0, _RPW), :],
                              rows_v, sem_g).wait()

        @pl.when(l + 1 < _L)
        def _():
            pltpu.async_copy(table_hbm.at[idxt_v.at[l + 1]], nxt_rows, sem_g)

        zero16 = jnp.zeros((16,), jnp.float32)
        for jrow in range(_RPW):
            lens = seq_v[pl.ds(jrow, 16)]
            mvec = (jnp.full((16,), 0, jnp.int32)
                    + (l < lens[0]).astype(jnp.int32)) > 0
            jsplat = jnp.full((16,), jrow, jnp.int32)
            for c4 in range(4):
                v = jnp.where(mvec, rows_v[jrow, pl.ds(c4 * 16, 16)], zero16)
                plsc.store_scatter(out_v, [dbase[c4], jsplat], v)
        pltpu.async_copy(out_v.at[:, pl.ds(0, _RPW)],
                         out_hbm.at[l, :, pl.ds(b0, _RPW)], sem_w)

    def body(j, carry):
        do_l(2 * j, rows_a, out_a, sem_w0, rows_b, j)
        do_l(2 * j + 1, rows_b, out_b, sem_w1, rows_a, j)
        return carry

    lax.fori_loop(0, _L // 2, body, 0)

    for sem_w, out_v in ((sem_w0, out_a), (sem_w1, out_b)):
        pltpu.make_async_copy(out_hbm.at[0, :, pl.ds(0, _RPW)],
                              out_v.at[:, pl.ds(0, _RPW)], sem_w).wait()


@jax.jit
def _emb_call(idx_flat, seq, table):
    mesh = plsc.VectorSubcoreMesh(core_axis_name="c", subcore_axis_name="s",
                                  num_cores=_NC, num_subcores=_NS)
    fn = pl.kernel(
        _emb_body,
        out_type=jax.ShapeDtypeStruct((_L, _D, _B), jnp.float32),
        mesh=mesh,
        scratch_types=[
            pltpu.VMEM((_RPW * _L,), jnp.int32),
            pltpu.VMEM((_L, _RPW), jnp.int32),
            pltpu.VMEM((_RPW + 16,), jnp.int32),
            pltpu.VMEM((_RPW, _D), jnp.float32),
            pltpu.VMEM((_RPW, _D), jnp.float32),
            pltpu.VMEM((_D, _SK), jnp.float32),
            pltpu.VMEM((_D, _SK), jnp.float32),
            pltpu.SemaphoreType.DMA,
            pltpu.SemaphoreType.DMA,
            pltpu.SemaphoreType.DMA,
        ],
        compiler_params=pltpu.CompilerParams(use_tc_tiling_on_sc=False,
                                             needs_layout_passes=False),
    )
    return fn(idx_flat, seq, table)


def kernel(indices, seq_lens, table):
    idx_flat = indices.reshape(_B * _L).astype(jnp.int32)
    seq = seq_lens.astype(jnp.int32)
    out_t = _emb_call(idx_flat, seq, table)
    out = jnp.transpose(out_t, (2, 0, 1))
    mask = (jnp.arange(_L, dtype=jnp.int32)[None, :]
            < seq_lens.astype(jnp.int32)[:, None]).astype(table.dtype)
    lengths = jnp.broadcast_to(mask[:, :, None], (_B, _L, _D))
    return out, lengths


# speedup vs baseline: 1.7082x; 1.7082x over previous
import functools

import jax
import jax.numpy as jnp
from jax import lax
from jax.experimental import pallas as pl
from jax.experimental.pallas import tpu as pltpu
from jax.experimental.pallas import tpu_sc as plsc

_B = 4096
_L = 200
_D = 64
_NC = 2
_NS = 16
_NW = _NC * _NS
_RPW = _B // _NW
_SK = _RPW + 1


def _emb_body(idx_hbm, seq_hbm, table_hbm,
              out_hbm,
              idxl_v, idxt_v, seq_v, rows_a, rows_b, out_a, out_b,
              sem_g, sem_w0, sem_w1):
    wid = lax.axis_index("s") * _NC + lax.axis_index("c")
    b0 = wid * _RPW
    pltpu.sync_copy(seq_hbm.at[pl.ds(b0, _RPW)], seq_v.at[pl.ds(0, _RPW)])
    pltpu.sync_copy(idx_hbm.at[pl.ds(b0 * _L, _RPW * _L)], idxl_v)

    iota16 = lax.iota(jnp.int32, 16)

    def tbody(l, carry):
        lsplat = jnp.full((16,), 0, jnp.int32) + l
        for jg in range(8):
            flat = (iota16 + (jg * 16)) * _L + lsplat
            vals = plsc.load_gather(idxl_v, [flat])
            idxt_v[l, pl.ds(jg * 16, 16)] = vals
        return carry
    lax.fori_loop(0, _L, tbody, 0)

    pltpu.async_copy(table_hbm.at[idxt_v.at[0]], rows_a, sem_g)

    dbase = [iota16 + (c4 * 16) for c4 in range(4)]

    def do_l(l, rows_v, out_v, sem_w, nxt_rows, j):
        @pl.when(j >= 1)
        def _():
            pltpu.make_async_copy(out_hbm.at[0, :, pl.ds(0, _RPW)],
                                  out_v.at[:, pl.ds(0, _RPW)], sem_w).wait()
        pltpu.make_async_copy(table_hbm.at[pl.ds(0, _RPW), :],
                              rows_v, sem_g).wait()

        @pl.when(l + 1 < _L)
        def _():
            pltpu.async_copy(table_hbm.at[idxt_v.at[l + 1]], nxt_rows, sem_g)

        zero16 = jnp.zeros((16,), jnp.float32)

        @functools.partial(plsc.parallel_loop, 0, _RPW, unroll=8)
        def _(jrow):
            lens = seq_v[pl.ds(jrow, 16)]
            mvec = (jnp.full((16,), 0, jnp.int32)
                    + (l < lens[0]).astype(jnp.int32)) > 0
            jsplat = jnp.full((16,), 0, jnp.int32) + jrow
            for c4 in range(4):
                v = jnp.where(mvec, rows_v[jrow, pl.ds(c4 * 16, 16)], zero16)
                plsc.store_scatter(out_v, [dbase[c4], jsplat], v)
        pltpu.async_copy(out_v.at[:, pl.ds(0, _RPW)],
                         out_hbm.at[l, :, pl.ds(b0, _RPW)], sem_w)

    def body(j, carry):
        do_l(2 * j, rows_a, out_a, sem_w0, rows_b, j)
        do_l(2 * j + 1, rows_b, out_b, sem_w1, rows_a, j)
        return carry

    lax.fori_loop(0, _L // 2, body, 0)

    for sem_w, out_v in ((sem_w0, out_a), (sem_w1, out_b)):
        pltpu.make_async_copy(out_hbm.at[0, :, pl.ds(0, _RPW)],
                              out_v.at[:, pl.ds(0, _RPW)], sem_w).wait()


@jax.jit
def _emb_call(idx_flat, seq, table):
    mesh = plsc.VectorSubcoreMesh(core_axis_name="c", subcore_axis_name="s",
                                  num_cores=_NC, num_subcores=_NS)
    fn = pl.kernel(
        _emb_body,
        out_type=jax.ShapeDtypeStruct((_L, _D, _B), jnp.float32),
        mesh=mesh,
        scratch_types=[
            pltpu.VMEM((_RPW * _L,), jnp.int32),
            pltpu.VMEM((_L, _RPW), jnp.int32),
            pltpu.VMEM((_RPW + 16,), jnp.int32),
            pltpu.VMEM((_RPW, _D), jnp.float32),
            pltpu.VMEM((_RPW, _D), jnp.float32),
            pltpu.VMEM((_D, _SK), jnp.float32),
            pltpu.VMEM((_D, _SK), jnp.float32),
            pltpu.SemaphoreType.DMA,
            pltpu.SemaphoreType.DMA,
            pltpu.SemaphoreType.DMA,
        ],
        compiler_params=pltpu.CompilerParams(use_tc_tiling_on_sc=False,
                                             needs_layout_passes=False),
    )
    return fn(idx_flat, seq, table)


def kernel(indices, seq_lens, table):
    idx_flat = indices.reshape(_B * _L).astype(jnp.int32)
    seq = seq_lens.astype(jnp.int32)
    out_t = _emb_call(idx_flat, seq, table)
    out = jnp.transpose(out_t, (2, 0, 1))
    mask = (jnp.arange(_L, dtype=jnp.int32)[None, :]
            < seq_lens.astype(jnp.int32)[:, None]).astype(table.dtype)
    lengths = jnp.broadcast_to(mask[:, :, None], (_B, _L, _D))
    return out, lengths
